# baseline (speedup 1.0000x reference)
import functools

import jax
import jax.numpy as jnp
from jax.experimental import pallas as pl
from jax.experimental.pallas import tpu as pltpu

_EPS = 1e-5
_THETA = 10000.0
_NH = 16
_NKV = 4
_REP = _NH // _NKV
_D = 128
_H = 2048
_I = 8192
_V = 32000
_B = 8
_S = 512
_M = _B * _S
_QKV_N = (_NH + 2 * _NKV) * _D
_ROPE_COLS = (_NH + _NKV) * _D

_VMEM_LIMIT = 56 * 1024 * 1024


def _cp(sem):
    return pltpu.CompilerParams(dimension_semantics=sem,
                                vmem_limit_bytes=_VMEM_LIMIT)


def _qkv_rope_kernel(x_ref, w_ref, cos_ref, sin_ref, o_ref, xb_ref, inv_ref,
                     *, tn):
    j = pl.program_id(1)

    @pl.when(j == 0)
    def _():
        xf = x_ref[...]
        inv_ref[...] = jax.lax.rsqrt(
            jnp.mean(xf * xf, axis=-1, keepdims=True) + _EPS)
        xb_ref[...] = xf.astype(jnp.bfloat16)

    w = w_ref[...].astype(jnp.bfloat16)
    out = jnp.dot(xb_ref[...], w,
                  preferred_element_type=jnp.float32) * inv_ref[...]

    cos = cos_ref[...]
    sin = sin_ref[...]
    half = _D // 2
    lane = jax.lax.broadcasted_iota(jnp.int32, (out.shape[0], _D), 1)
    sign = jnp.where(lane < half, -1.0, 1.0)
    for h in range(tn // _D):
        c0 = h * _D
        xh = out[:, c0:c0 + _D]
        rot = pltpu.roll(xh, shift=half, axis=1) * sign
        roped = xh * cos + rot * sin
        do_rope = (j * tn + c0) < _ROPE_COLS
        o_ref[:, c0:c0 + _D] = jnp.where(do_rope, roped, xh).astype(o_ref.dtype)


def _qkv_rope(x2d, w, cos, sin, *, tn=1024):
    grid = (_M // _S, _QKV_N // tn)
    return pl.pallas_call(
        functools.partial(_qkv_rope_kernel, tn=tn),
        out_shape=jax.ShapeDtypeStruct((_M, _QKV_N), jnp.bfloat16),
        grid=grid,
        in_specs=[
            pl.BlockSpec((_S, _H), lambda i, j: (i, 0)),
            pl.BlockSpec((_H, tn), lambda i, j: (0, j)),
            pl.BlockSpec((_S, _D), lambda i, j: (0, 0)),
            pl.BlockSpec((_S, _D), lambda i, j: (0, 0)),
        ],
        out_specs=pl.BlockSpec((_S, tn), lambda i, j: (i, j)),
        scratch_shapes=[pltpu.VMEM((_S, _H), jnp.bfloat16),
                        pltpu.VMEM((_S, 1), jnp.float32)],
        compiler_params=_cp(("parallel", "arbitrary")),
    )(x2d, w, cos, sin)


def _attn_kernel(q_ref, k_ref, v_ref, o_ref):
    k = k_ref[...]
    v = v_ref[...]
    for h in range(_REP):
        c0 = h * _D
        q = q_ref[:, c0:c0 + _D]
        s = jax.lax.dot_general(q, k, (((1,), (1,)), ((), ())),
                                preferred_element_type=jnp.float32)
        m = jnp.max(s, axis=-1, keepdims=True)
        p = jnp.exp(s - m)
        l = jnp.sum(p, axis=-1, keepdims=True)
        o = jnp.dot(p.astype(jnp.bfloat16), v, preferred_element_type=jnp.float32)
        o_ref[:, c0:c0 + _D] = (o / l).astype(o_ref.dtype)


def _attention(qkv):
    grid = (_B, _NKV)
    return pl.pallas_call(
        _attn_kernel,
        out_shape=jax.ShapeDtypeStruct((_M, _NH * _D), jnp.bfloat16),
        grid=grid,
        in_specs=[
            pl.BlockSpec((_S, _REP * _D), lambda b, g: (b, g)),
            pl.BlockSpec((_S, _D), lambda b, g: (b, _NH + g)),
            pl.BlockSpec((_S, _D), lambda b, g: (b, _NH + _NKV + g)),
        ],
        out_specs=pl.BlockSpec((_S, _REP * _D), lambda b, g: (b, g)),
        compiler_params=_cp(("parallel", "parallel")),
    )(qkv, qkv, qkv)


def _oproj_kernel(a_ref, w_ref, r_ref, o_ref):
    w = w_ref[...].astype(jnp.bfloat16)
    acc = jnp.dot(a_ref[...], w, preferred_element_type=jnp.float32)
    o_ref[...] = (r_ref[...] + acc).astype(o_ref.dtype)


def _oproj_residual(attn, w, residual, *, tm=512, tn=1024):
    grid = (_M // tm, _H // tn)
    return pl.pallas_call(
        _oproj_kernel,
        out_shape=jax.ShapeDtypeStruct((_M, _H), jnp.float32),
        grid=grid,
        in_specs=[
            pl.BlockSpec((tm, _H), lambda i, j: (i, 0)),
            pl.BlockSpec((_H, tn), lambda i, j: (0, j)),
            pl.BlockSpec((tm, tn), lambda i, j: (i, j)),
        ],
        out_specs=pl.BlockSpec((tm, tn), lambda i, j: (i, j)),
        compiler_params=_cp(("parallel", "parallel")),
    )(attn, w, residual)


def _mlp_kernel(x_ref, wg_ref, wu_ref, wd_ref, o_ref, xn_ref, acc_ref):
    l = pl.program_id(1)

    @pl.when(l == 0)
    def _():
        xf = x_ref[...]
        inv = jax.lax.rsqrt(jnp.mean(xf * xf, axis=-1, keepdims=True) + _EPS)
        xn_ref[...] = (xf * inv).astype(xn_ref.dtype)
        acc_ref[...] = jnp.zeros_like(acc_ref)

    xn = xn_ref[...]
    wg = wg_ref[...].astype(jnp.bfloat16)
    wu = wu_ref[...].astype(jnp.bfloat16)
    g = jnp.dot(xn, wg, preferred_element_type=jnp.float32)
    u = jnp.dot(xn, wu, preferred_element_type=jnp.float32)
    h = (g * jax.nn.sigmoid(g)) * u
    acc_ref[...] += jnp.dot(h.astype(jnp.bfloat16), wd_ref[...].astype(jnp.bfloat16),
                            preferred_element_type=jnp.float32)

    @pl.when(l == pl.num_programs(1) - 1)
    def _():
        o_ref[...] = (x_ref[...] + acc_ref[...]).astype(o_ref.dtype)


def _mlp_residual(x2d, wg, wu, wd, *, tm=512, ti=512):
    grid = (_M // tm, _I // ti)
    return pl.pallas_call(
        _mlp_kernel,
        out_shape=jax.ShapeDtypeStruct((_M, _H), jnp.float32),
        grid=grid,
        in_specs=[
            pl.BlockSpec((tm, _H), lambda i, l: (i, 0)),
            pl.BlockSpec((_H, ti), lambda i, l: (0, l)),
            pl.BlockSpec((_H, ti), lambda i, l: (0, l)),
            pl.BlockSpec((ti, _H), lambda i, l: (l, 0)),
        ],
        out_specs=pl.BlockSpec((tm, _H), lambda i, l: (i, 0)),
        scratch_shapes=[pltpu.VMEM((tm, _H), jnp.bfloat16),
                        pltpu.VMEM((tm, _H), jnp.float32)],
        compiler_params=_cp(("parallel", "arbitrary")),
    )(x2d, wg, wu, wd)


def _head_kernel(x_ref, w_ref, o_ref, xb_ref, inv_ref):
    j = pl.program_id(1)

    @pl.when(j == 0)
    def _():
        xf = x_ref[...]
        inv_ref[...] = jax.lax.rsqrt(
            jnp.mean(xf * xf, axis=-1, keepdims=True) + _EPS)
        xb_ref[...] = xf.astype(jnp.bfloat16)

    w = w_ref[...].astype(jnp.bfloat16)
    acc = jnp.dot(xb_ref[...], w, preferred_element_type=jnp.float32)
    o_ref[...] = (acc * inv_ref[...]).astype(o_ref.dtype)


def _rms_head(x2d, w, *, tm=512, tn=1280):
    grid = (_M // tm, _V // tn)
    return pl.pallas_call(
        _head_kernel,
        out_shape=jax.ShapeDtypeStruct((_M, _V), jnp.float32),
        grid=grid,
        in_specs=[
            pl.BlockSpec((tm, _H), lambda i, j: (i, 0)),
            pl.BlockSpec((_H, tn), lambda i, j: (0, j)),
        ],
        out_specs=pl.BlockSpec((tm, tn), lambda i, j: (i, j)),
        scratch_shapes=[pltpu.VMEM((tm, _H), jnp.bfloat16),
                        pltpu.VMEM((tm, 1), jnp.float32)],
        compiler_params=_cp(("parallel", "arbitrary")),
    )(x2d, w)


def _rope_tables():
    inv_freq = 1.0 / _THETA ** (
        jnp.arange(0, _D, 2, dtype=jnp.float32) / _D)
    pos = jnp.arange(_S, dtype=jnp.float32)
    freqs = pos[:, None] * inv_freq[None, :]
    emb = jnp.concatenate([freqs, freqs], axis=-1)
    return jnp.cos(emb), jnp.sin(emb)


def _layer(x2d, cos, sin, wqkv, wo, wg, wu, wd):
    qkv = _qkv_rope(x2d, wqkv, cos, sin)
    attn = _attention(qkv)
    x2d = _oproj_residual(attn, wo, x2d)
    return _mlp_residual(x2d, wg, wu, wd)


def kernel(input_ids, embed, lm_head_fused,
           layer0_wqkv_fused, layer0_wo, layer0_wg_fused, layer0_wu_fused,
           layer0_wd,
           layer1_wqkv_fused, layer1_wo, layer1_wg_fused, layer1_wu_fused,
           layer1_wd):
    x = jnp.take(embed, input_ids, axis=0)
    x2d = x.reshape(_M, _H)
    cos, sin = _rope_tables()
    x2d = _layer(x2d, cos, sin, layer0_wqkv_fused, layer0_wo,
                 layer0_wg_fused, layer0_wu_fused, layer0_wd)
    x2d = _layer(x2d, cos, sin, layer1_wqkv_fused, layer1_wo,
                 layer1_wg_fused, layer1_wu_fused, layer1_wd)
    logits = _rms_head(x2d, lm_head_fused)
    return logits.reshape(_B, _S, _V)


# speedup vs baseline: 1.2318x; 1.2318x over previous
import functools

import jax
import jax.numpy as jnp
from jax.experimental import pallas as pl
from jax.experimental.pallas import tpu as pltpu

_EPS = 1e-5
_THETA = 10000.0
_NH = 16
_NKV = 4
_REP = _NH // _NKV
_D = 128
_H = 2048
_I = 8192
_V = 32000
_B = 8
_S = 512
_M = _B * _S
_QKV_N = (_NH + 2 * _NKV) * _D
_ROPE_COLS = (_NH + _NKV) * _D

_VMEM_LIMIT = 58 * 1024 * 1024


def _cp(sem):
    return pltpu.CompilerParams(dimension_semantics=sem,
                                vmem_limit_bytes=_VMEM_LIMIT)


def _qkv_rope_kernel(x_ref, w_ref, cos_ref, sin_ref, o_ref, xb_ref, inv_ref,
                     *, tn):
    j = pl.program_id(1)

    @pl.when(j == 0)
    def _():
        xf = x_ref[...]
        inv_ref[...] = jax.lax.rsqrt(
            jnp.mean(xf * xf, axis=-1, keepdims=True) + _EPS)
        xb_ref[...] = xf.astype(jnp.bfloat16)

    w = w_ref[...].astype(jnp.bfloat16)
    out = jnp.dot(xb_ref[...], w,
                  preferred_element_type=jnp.float32) * inv_ref[...]

    cos = cos_ref[...]
    sin = sin_ref[...]
    half = _D // 2
    lane = jax.lax.broadcasted_iota(jnp.int32, (out.shape[0], _D), 1)
    sign = jnp.where(lane < half, -1.0, 1.0)
    for h in range(tn // _D):
        c0 = h * _D
        xh = out[:, c0:c0 + _D]
        rot = pltpu.roll(xh, shift=half, axis=1) * sign
        roped = xh * cos + rot * sin
        do_rope = (j * tn + c0) < _ROPE_COLS
        o_ref[:, c0:c0 + _D] = jnp.where(do_rope, roped, xh).astype(o_ref.dtype)


def _qkv_rope(x2d, w, cos, sin, *, tm=1024, tn=1024):
    cos_t = jnp.tile(cos, (tm // _S, 1))
    sin_t = jnp.tile(sin, (tm // _S, 1))
    grid = (_M // tm, _QKV_N // tn)
    return pl.pallas_call(
        functools.partial(_qkv_rope_kernel, tn=tn),
        out_shape=jax.ShapeDtypeStruct((_M, _QKV_N), jnp.bfloat16),
        grid=grid,
        in_specs=[
            pl.BlockSpec((tm, _H), lambda i, j: (i, 0)),
            pl.BlockSpec((_H, tn), lambda i, j: (0, j)),
            pl.BlockSpec((tm, _D), lambda i, j: (0, 0)),
            pl.BlockSpec((tm, _D), lambda i, j: (0, 0)),
        ],
        out_specs=pl.BlockSpec((tm, tn), lambda i, j: (i, j)),
        scratch_shapes=[pltpu.VMEM((tm, _H), jnp.bfloat16),
                        pltpu.VMEM((tm, 1), jnp.float32)],
        compiler_params=_cp(("parallel", "arbitrary")),
    )(x2d, w, cos_t, sin_t)


def _attn_kernel(q_ref, k_ref, v_ref, o_ref):
    k = k_ref[...]
    v = v_ref[...]
    for h in range(_REP):
        c0 = h * _D
        q = q_ref[:, c0:c0 + _D]
        s = jax.lax.dot_general(q, k, (((1,), (1,)), ((), ())),
                                preferred_element_type=jnp.float32)
        m = jnp.max(s, axis=-1, keepdims=True)
        p = jnp.exp(s - m)
        l = jnp.sum(p, axis=-1, keepdims=True)
        o = jnp.dot(p.astype(jnp.bfloat16), v, preferred_element_type=jnp.float32)
        o_ref[:, c0:c0 + _D] = (o / l).astype(o_ref.dtype)


def _attention(qkv):
    grid = (_B, _NKV)
    return pl.pallas_call(
        _attn_kernel,
        out_shape=jax.ShapeDtypeStruct((_M, _NH * _D), jnp.bfloat16),
        grid=grid,
        in_specs=[
            pl.BlockSpec((_S, _REP * _D), lambda b, g: (b, g)),
            pl.BlockSpec((_S, _D), lambda b, g: (b, _NH + g)),
            pl.BlockSpec((_S, _D), lambda b, g: (b, _NH + _NKV + g)),
        ],
        out_specs=pl.BlockSpec((_S, _REP * _D), lambda b, g: (b, g)),
        compiler_params=_cp(("parallel", "parallel")),
    )(qkv, qkv, qkv)


def _oproj_kernel(a_ref, w_ref, r_ref, o_ref):
    w = w_ref[...].astype(jnp.bfloat16)
    acc = jnp.dot(a_ref[...], w, preferred_element_type=jnp.float32)
    o_ref[...] = (r_ref[...] + acc).astype(o_ref.dtype)


def _oproj_residual(attn, w, residual, *, tm=1024, tn=1024):
    grid = (_M // tm, _H // tn)
    return pl.pallas_call(
        _oproj_kernel,
        out_shape=jax.ShapeDtypeStruct((_M, _H), jnp.float32),
        grid=grid,
        in_specs=[
            pl.BlockSpec((tm, _H), lambda i, j: (i, 0)),
            pl.BlockSpec((_H, tn), lambda i, j: (0, j)),
            pl.BlockSpec((tm, tn), lambda i, j: (i, j)),
        ],
        out_specs=pl.BlockSpec((tm, tn), lambda i, j: (i, j)),
        compiler_params=_cp(("parallel", "parallel")),
    )(attn, w, residual)


def _mlp_kernel(x_ref, wg_ref, wu_ref, wd_ref, o_ref, xn_ref, acc_ref):
    l = pl.program_id(1)

    @pl.when(l == 0)
    def _():
        xf = x_ref[...]
        inv = jax.lax.rsqrt(jnp.mean(xf * xf, axis=-1, keepdims=True) + _EPS)
        xn_ref[...] = (xf * inv).astype(xn_ref.dtype)
        acc_ref[...] = jnp.zeros_like(acc_ref)

    xn = xn_ref[...]
    wg = wg_ref[...].astype(jnp.bfloat16)
    wu = wu_ref[...].astype(jnp.bfloat16)
    g = jnp.dot(xn, wg, preferred_element_type=jnp.float32)
    u = jnp.dot(xn, wu, preferred_element_type=jnp.float32)
    h = (g * jax.nn.sigmoid(g)) * u
    acc_ref[...] += jnp.dot(h.astype(jnp.bfloat16), wd_ref[...].astype(jnp.bfloat16),
                            preferred_element_type=jnp.float32)

    @pl.when(l == pl.num_programs(1) - 1)
    def _():
        o_ref[...] = (x_ref[...] + acc_ref[...]).astype(o_ref.dtype)


def _mlp_residual(x2d, wg, wu, wd, *, tm=1024, ti=256):
    grid = (_M // tm, _I // ti)
    return pl.pallas_call(
        _mlp_kernel,
        out_shape=jax.ShapeDtypeStruct((_M, _H), jnp.float32),
        grid=grid,
        in_specs=[
            pl.BlockSpec((tm, _H), lambda i, l: (i, 0)),
            pl.BlockSpec((_H, ti), lambda i, l: (0, l)),
            pl.BlockSpec((_H, ti), lambda i, l: (0, l)),
            pl.BlockSpec((ti, _H), lambda i, l: (l, 0)),
        ],
        out_specs=pl.BlockSpec((tm, _H), lambda i, l: (i, 0)),
        scratch_shapes=[pltpu.VMEM((tm, _H), jnp.bfloat16),
                        pltpu.VMEM((tm, _H), jnp.float32)],
        compiler_params=_cp(("parallel", "arbitrary")),
    )(x2d, wg, wu, wd)


def _norm_cast_kernel(x_ref, xb_ref, inv_ref):
    xf = x_ref[...]
    inv = jax.lax.rsqrt(jnp.mean(xf * xf, axis=-1, keepdims=True) + _EPS)
    xb_ref[...] = xf.astype(xb_ref.dtype)
    inv_ref[...] = jnp.broadcast_to(inv, inv_ref.shape)


def _norm_cast(x2d, *, tm=512):
    grid = (_M // tm,)
    return pl.pallas_call(
        _norm_cast_kernel,
        out_shape=(jax.ShapeDtypeStruct((_M, _H), jnp.bfloat16),
                   jax.ShapeDtypeStruct((_M, 128), jnp.float32)),
        grid=grid,
        in_specs=[pl.BlockSpec((tm, _H), lambda i: (i, 0))],
        out_specs=(pl.BlockSpec((tm, _H), lambda i: (i, 0)),
                   pl.BlockSpec((tm, 128), lambda i: (i, 0))),
        compiler_params=_cp(("parallel",)),
    )(x2d)


def _head_kernel(xb_ref, inv_ref, w_ref, o_ref):
    w = w_ref[...].astype(jnp.bfloat16)
    acc = jnp.dot(xb_ref[...], w, preferred_element_type=jnp.float32)
    o_ref[...] = (acc * inv_ref[:, :1]).astype(o_ref.dtype)


def _rms_head(x2d, w, *, tn=256):
    xb, inv = _norm_cast(x2d)
    grid = (_V // tn,)
    return pl.pallas_call(
        _head_kernel,
        out_shape=jax.ShapeDtypeStruct((_M, _V), jnp.float32),
        grid=grid,
        in_specs=[
            pl.BlockSpec((_M, _H), lambda j: (0, 0)),
            pl.BlockSpec((_M, 128), lambda j: (0, 0)),
            pl.BlockSpec((_H, tn), lambda j: (0, j)),
        ],
        out_specs=pl.BlockSpec((_M, tn), lambda j: (0, j)),
        compiler_params=_cp(("parallel",)),
    )(xb, inv, w)


def _rope_tables():
    inv_freq = 1.0 / _THETA ** (
        jnp.arange(0, _D, 2, dtype=jnp.float32) / _D)
    pos = jnp.arange(_S, dtype=jnp.float32)
    freqs = pos[:, None] * inv_freq[None, :]
    emb = jnp.concatenate([freqs, freqs], axis=-1)
    return jnp.cos(emb), jnp.sin(emb)


def _layer(x2d, cos, sin, wqkv, wo, wg, wu, wd):
    qkv = _qkv_rope(x2d, wqkv, cos, sin)
    attn = _attention(qkv)
    x2d = _oproj_residual(attn, wo, x2d)
    return _mlp_residual(x2d, wg, wu, wd)


def kernel(input_ids, embed, lm_head_fused,
           layer0_wqkv_fused, layer0_wo, layer0_wg_fused, layer0_wu_fused,
           layer0_wd,
           layer1_wqkv_fused, layer1_wo, layer1_wg_fused, layer1_wu_fused,
           layer1_wd):
    x = jnp.take(embed, input_ids, axis=0)
    x2d = x.reshape(_M, _H)
    cos, sin = _rope_tables()
    x2d = _layer(x2d, cos, sin, layer0_wqkv_fused, layer0_wo,
                 layer0_wg_fused, layer0_wu_fused, layer0_wd)
    x2d = _layer(x2d, cos, sin, layer1_wqkv_fused, layer1_wo,
                 layer1_wg_fused, layer1_wu_fused, layer1_wd)
    logits = _rms_head(x2d, lm_head_fused)
    return logits.reshape(_B, _S, _V)


# speedup vs baseline: 1.2364x; 1.0038x over previous
import functools

import jax
import jax.numpy as jnp
from jax.experimental import pallas as pl
from jax.experimental.pallas import tpu as pltpu

_EPS = 1e-5
_THETA = 10000.0
_NH = 16
_NKV = 4
_REP = _NH // _NKV
_D = 128
_H = 2048
_I = 8192
_V = 32000
_B = 8
_S = 512
_M = _B * _S
_QKV_N = (_NH + 2 * _NKV) * _D
_ROPE_COLS = (_NH + _NKV) * _D

_VMEM_LIMIT = 58 * 1024 * 1024


def _cp(sem):
    return pltpu.CompilerParams(dimension_semantics=sem,
                                vmem_limit_bytes=_VMEM_LIMIT)


def _qkv_rope_kernel(x_ref, w_ref, cos_ref, sin_ref, o_ref, xb_ref, inv_ref,
                     *, tn):
    j = pl.program_id(1)

    @pl.when(j == 0)
    def _():
        xf = x_ref[...]
        inv_ref[...] = jax.lax.rsqrt(
            jnp.mean(xf * xf, axis=-1, keepdims=True) + _EPS)
        xb_ref[...] = xf.astype(jnp.bfloat16)

    w = w_ref[...].astype(jnp.bfloat16)
    out = jnp.dot(xb_ref[...], w,
                  preferred_element_type=jnp.float32) * inv_ref[...]

    cos = cos_ref[...]
    sin = sin_ref[...]
    half = _D // 2
    lane = jax.lax.broadcasted_iota(jnp.int32, (out.shape[0], _D), 1)
    sign = jnp.where(lane < half, -1.0, 1.0)
    for h in range(tn // _D):
        c0 = h * _D
        xh = out[:, c0:c0 + _D]
        rot = pltpu.roll(xh, shift=half, axis=1) * sign
        roped = xh * cos + rot * sin
        do_rope = (j * tn + c0) < _ROPE_COLS
        o_ref[:, c0:c0 + _D] = jnp.where(do_rope, roped, xh).astype(o_ref.dtype)


def _qkv_rope(x2d, w, cos, sin, *, tm=1024, tn=1024):
    cos_t = jnp.tile(cos, (tm // _S, 1))
    sin_t = jnp.tile(sin, (tm // _S, 1))
    grid = (_M // tm, _QKV_N // tn)
    return pl.pallas_call(
        functools.partial(_qkv_rope_kernel, tn=tn),
        out_shape=jax.ShapeDtypeStruct((_M, _QKV_N), jnp.bfloat16),
        grid=grid,
        in_specs=[
            pl.BlockSpec((tm, _H), lambda i, j: (i, 0)),
            pl.BlockSpec((_H, tn), lambda i, j: (0, j)),
            pl.BlockSpec((tm, _D), lambda i, j: (0, 0)),
            pl.BlockSpec((tm, _D), lambda i, j: (0, 0)),
        ],
        out_specs=pl.BlockSpec((tm, tn), lambda i, j: (i, j)),
        scratch_shapes=[pltpu.VMEM((tm, _H), jnp.bfloat16),
                        pltpu.VMEM((tm, 1), jnp.float32)],
        compiler_params=_cp(("parallel", "arbitrary")),
    )(x2d, w, cos_t, sin_t)


def _attn_kernel(q_ref, k_ref, v_ref, o_ref):
    k = k_ref[...]
    v = v_ref[...]
    for h in range(_REP):
        c0 = h * _D
        q = q_ref[:, c0:c0 + _D]
        s = jax.lax.dot_general(q, k, (((1,), (1,)), ((), ())),
                                preferred_element_type=jnp.float32)
        m = jnp.max(s, axis=-1, keepdims=True)
        p = jnp.exp(s - m)
        l = jnp.sum(p, axis=-1, keepdims=True)
        o = jnp.dot(p.astype(jnp.bfloat16), v, preferred_element_type=jnp.float32)
        o_ref[:, c0:c0 + _D] = (o / l).astype(o_ref.dtype)


def _attention(qkv):
    grid = (_B, _NKV)
    return pl.pallas_call(
        _attn_kernel,
        out_shape=jax.ShapeDtypeStruct((_M, _NH * _D), jnp.bfloat16),
        grid=grid,
        in_specs=[
            pl.BlockSpec((_S, _REP * _D), lambda b, g: (b, g)),
            pl.BlockSpec((_S, _D), lambda b, g: (b, _NH + g)),
            pl.BlockSpec((_S, _D), lambda b, g: (b, _NH + _NKV + g)),
        ],
        out_specs=pl.BlockSpec((_S, _REP * _D), lambda b, g: (b, g)),
        compiler_params=_cp(("parallel", "parallel")),
    )(qkv, qkv, qkv)


def _oproj_norm_kernel(a_ref, w_ref, r_ref, x2_ref, xn_ref):
    w = w_ref[...].astype(jnp.bfloat16)
    acc = jnp.dot(a_ref[...], w, preferred_element_type=jnp.float32)
    x2 = r_ref[...] + acc
    x2_ref[...] = x2
    inv = jax.lax.rsqrt(jnp.mean(x2 * x2, axis=-1, keepdims=True) + _EPS)
    xn_ref[...] = (x2 * inv).astype(xn_ref.dtype)


def _oproj_norm(attn, w, residual, *, tm=512):
    grid = (_M // tm,)
    return pl.pallas_call(
        _oproj_norm_kernel,
        out_shape=(jax.ShapeDtypeStruct((_M, _H), jnp.float32),
                   jax.ShapeDtypeStruct((_M, _H), jnp.bfloat16)),
        grid=grid,
        in_specs=[
            pl.BlockSpec((tm, _H), lambda i: (i, 0)),
            pl.BlockSpec((_H, _H), lambda i: (0, 0)),
            pl.BlockSpec((tm, _H), lambda i: (i, 0)),
        ],
        out_specs=(pl.BlockSpec((tm, _H), lambda i: (i, 0)),
                   pl.BlockSpec((tm, _H), lambda i: (i, 0))),
        compiler_params=_cp(("parallel",)),
    )(attn, w, residual)


def _gateup_kernel(xn_ref, wg_ref, wu_ref, h_ref):
    xn = xn_ref[...]
    wg = wg_ref[...].astype(jnp.bfloat16)
    wu = wu_ref[...].astype(jnp.bfloat16)
    g = jnp.dot(xn, wg, preferred_element_type=jnp.float32)
    u = jnp.dot(xn, wu, preferred_element_type=jnp.float32)
    h_ref[...] = ((g * jax.nn.sigmoid(g)) * u).astype(h_ref.dtype)


def _gateup(xn, wg, wu, *, ti=256):
    grid = (_I // ti,)
    return pl.pallas_call(
        _gateup_kernel,
        out_shape=jax.ShapeDtypeStruct((_M, _I), jnp.bfloat16),
        grid=grid,
        in_specs=[
            pl.BlockSpec((_M, _H), lambda l: (0, 0)),
            pl.BlockSpec((_H, ti), lambda l: (0, l)),
            pl.BlockSpec((_H, ti), lambda l: (0, l)),
        ],
        out_specs=pl.BlockSpec((_M, ti), lambda l: (0, l)),
        compiler_params=_cp(("parallel",)),
    )(xn, wg, wu)


def _down_kernel(h_ref, wd_ref, r_ref, o_ref):
    kk = pl.program_id(2)
    wd = wd_ref[...].astype(jnp.bfloat16)
    d = jnp.dot(h_ref[...], wd, preferred_element_type=jnp.float32)

    @pl.when(kk == 0)
    def _():
        o_ref[...] = r_ref[...] + d

    @pl.when(kk != 0)
    def _():
        o_ref[...] = o_ref[...] + d


def _down_residual(h, wd, residual, *, tm=1024, tn=1024, tk=2048):
    grid = (_M // tm, _H // tn, _I // tk)
    return pl.pallas_call(
        _down_kernel,
        out_shape=jax.ShapeDtypeStruct((_M, _H), jnp.float32),
        grid=grid,
        in_specs=[
            pl.BlockSpec((tm, tk), lambda i, jn, kk: (i, kk)),
            pl.BlockSpec((tk, tn), lambda i, jn, kk: (kk, jn)),
            pl.BlockSpec((tm, tn), lambda i, jn, kk: (i, jn)),
        ],
        out_specs=pl.BlockSpec((tm, tn), lambda i, jn, kk: (i, jn)),
        compiler_params=_cp(("parallel", "parallel", "arbitrary")),
    )(h, wd, residual)


def _norm_cast_kernel(x_ref, xb_ref, inv_ref):
    xf = x_ref[...]
    inv = jax.lax.rsqrt(jnp.mean(xf * xf, axis=-1, keepdims=True) + _EPS)
    xb_ref[...] = xf.astype(xb_ref.dtype)
    inv_ref[...] = jnp.broadcast_to(inv, inv_ref.shape)


def _norm_cast(x2d, *, tm=512):
    grid = (_M // tm,)
    return pl.pallas_call(
        _norm_cast_kernel,
        out_shape=(jax.ShapeDtypeStruct((_M, _H), jnp.bfloat16),
                   jax.ShapeDtypeStruct((_M, 128), jnp.float32)),
        grid=grid,
        in_specs=[pl.BlockSpec((tm, _H), lambda i: (i, 0))],
        out_specs=(pl.BlockSpec((tm, _H), lambda i: (i, 0)),
                   pl.BlockSpec((tm, 128), lambda i: (i, 0))),
        compiler_params=_cp(("parallel",)),
    )(x2d)


def _head_kernel(xb_ref, inv_ref, w_ref, o_ref):
    w = w_ref[...].astype(jnp.bfloat16)
    acc = jnp.dot(xb_ref[...], w, preferred_element_type=jnp.float32)
    o_ref[...] = (acc * inv_ref[:, :1]).astype(o_ref.dtype)


def _rms_head(x2d, w, *, tn=256):
    xb, inv = _norm_cast(x2d)
    grid = (_V // tn,)
    return pl.pallas_call(
        _head_kernel,
        out_shape=jax.ShapeDtypeStruct((_M, _V), jnp.float32),
        grid=grid,
        in_specs=[
            pl.BlockSpec((_M, _H), lambda j: (0, 0)),
            pl.BlockSpec((_M, 128), lambda j: (0, 0)),
            pl.BlockSpec((_H, tn), lambda j: (0, j)),
        ],
        out_specs=pl.BlockSpec((_M, tn), lambda j: (0, j)),
        compiler_params=_cp(("parallel",)),
    )(xb, inv, w)


def _rope_tables():
    inv_freq = 1.0 / _THETA ** (
        jnp.arange(0, _D, 2, dtype=jnp.float32) / _D)
    pos = jnp.arange(_S, dtype=jnp.float32)
    freqs = pos[:, None] * inv_freq[None, :]
    emb = jnp.concatenate([freqs, freqs], axis=-1)
    return jnp.cos(emb), jnp.sin(emb)


def _layer(x2d, cos, sin, wqkv, wo, wg, wu, wd):
    qkv = _qkv_rope(x2d, wqkv, cos, sin)
    attn = _attention(qkv)
    x2, xn = _oproj_norm(attn, wo, x2d)
    h = _gateup(xn, wg, wu)
    return _down_residual(h, wd, x2)


def kernel(input_ids, embed, lm_head_fused,
           layer0_wqkv_fused, layer0_wo, layer0_wg_fused, layer0_wu_fused,
           layer0_wd,
           layer1_wqkv_fused, layer1_wo, layer1_wg_fused, layer1_wu_fused,
           layer1_wd):
    x = jnp.take(embed, input_ids, axis=0)
    x2d = x.reshape(_M, _H)
    cos, sin = _rope_tables()
    x2d = _layer(x2d, cos, sin, layer0_wqkv_fused, layer0_wo,
                 layer0_wg_fused, layer0_wu_fused, layer0_wd)
    x2d = _layer(x2d, cos, sin, layer1_wqkv_fused, layer1_wo,
                 layer1_wg_fused, layer1_wu_fused, layer1_wd)
    logits = _rms_head(x2d, lm_head_fused)
    return logits.reshape(_B, _S, _V)
